# baseline (speedup 1.0000x reference)
import jax
import jax.numpy as jnp
from jax.experimental import pallas as pl
from jax.experimental.pallas import tpu as pltpu
from functools import partial

N = 50000
E = 800000
H = 128
L = 4
EDGE_IN = 2
MASK_RATIO = 0.15

BE = 4000
BN = 2000


def _msg_body(hs_ref, hd_ref, attr_ref, ws_ref, wd_ref, we_ref, bm_ref,
              wedge_ref, bedge_ref, m_ref):
    e = jnp.maximum(attr_ref[...] @ wedge_ref[...] + bedge_ref[...], 0.0)
    acc = hs_ref[...] @ ws_ref[...]
    acc = acc + hd_ref[...] @ wd_ref[...]
    acc = acc + e @ we_ref[...]
    m_ref[...] = jnp.maximum(acc + bm_ref[...], 0.0)


def _msg_matmul(hs, hd, attr, Ws, Wd, We, bm, Wedge, bedge):
    grid = (E // BE,)
    return pl.pallas_call(
        _msg_body,
        grid=grid,
        in_specs=[
            pl.BlockSpec((BE, H), lambda i: (i, 0)),
            pl.BlockSpec((BE, H), lambda i: (i, 0)),
            pl.BlockSpec((BE, EDGE_IN), lambda i: (i, 0)),
            pl.BlockSpec((H, H), lambda i: (0, 0)),
            pl.BlockSpec((H, H), lambda i: (0, 0)),
            pl.BlockSpec((H, H), lambda i: (0, 0)),
            pl.BlockSpec((1, H), lambda i: (0, 0)),
            pl.BlockSpec((EDGE_IN, H), lambda i: (0, 0)),
            pl.BlockSpec((1, H), lambda i: (0, 0)),
        ],
        out_specs=pl.BlockSpec((BE, H), lambda i: (i, 0)),
        out_shape=jax.ShapeDtypeStruct((E, H), jnp.float32),
        compiler_params=pltpu.CompilerParams(
            dimension_semantics=("arbitrary",)),
    )(hs, hd, attr, Ws, Wd, We, bm, Wedge, bedge)


def _upd_body(h_ref, agg_ref, wu1_ref, wu2_ref, bu_ref, out_ref):
    acc = h_ref[...] @ wu1_ref[...]
    acc = acc + agg_ref[...] @ wu2_ref[...]
    out_ref[...] = h_ref[...] + jnp.maximum(acc + bu_ref[...], 0.0)


def _update(h, agg, Wu1, Wu2, bu, n_pad):
    grid = (n_pad // BN,)
    return pl.pallas_call(
        _upd_body,
        grid=grid,
        in_specs=[
            pl.BlockSpec((BN, H), lambda i: (i, 0)),
            pl.BlockSpec((BN, H), lambda i: (i, 0)),
            pl.BlockSpec((H, H), lambda i: (0, 0)),
            pl.BlockSpec((H, H), lambda i: (0, 0)),
            pl.BlockSpec((1, H), lambda i: (0, 0)),
        ],
        out_specs=pl.BlockSpec((BN, H), lambda i: (i, 0)),
        out_shape=jax.ShapeDtypeStruct((n_pad, H), jnp.float32),
        compiler_params=pltpu.CompilerParams(
            dimension_semantics=("arbitrary",)),
    )(h, agg, Wu1, Wu2, bu)


def _head_body(a_ref, b_ref, b1_ref, w2_ref, b2_ref, out_ref):
    hid = jnp.maximum(a_ref[...] + b_ref[...] + b1_ref[...], 0.0)
    out_ref[...] = hid @ w2_ref[...] + b2_ref[...]


def _head(a_s, b_d, b1, W2, b2):
    grid = (E // BE,)
    return pl.pallas_call(
        _head_body,
        grid=grid,
        in_specs=[
            pl.BlockSpec((BE, H), lambda i: (i, 0)),
            pl.BlockSpec((BE, H), lambda i: (i, 0)),
            pl.BlockSpec((1, H), lambda i: (0, 0)),
            pl.BlockSpec((H, EDGE_IN), lambda i: (0, 0)),
            pl.BlockSpec((1, EDGE_IN), lambda i: (0, 0)),
        ],
        out_specs=pl.BlockSpec((BE, EDGE_IN), lambda i: (i, 0)),
        out_shape=jax.ShapeDtypeStruct((E, EDGE_IN), jnp.float32),
        compiler_params=pltpu.CompilerParams(
            dimension_semantics=("arbitrary",)),
    )(a_s, b_d, b1, W2, b2)


def _proj_body(h_ref, w1a_ref, w1b_ref, outa_ref, outb_ref):
    outa_ref[...] = h_ref[...] @ w1a_ref[...]
    outb_ref[...] = h_ref[...] @ w1b_ref[...]


def _proj(node_emb, W1a, W1b, n_pad):
    grid = (n_pad // BN,)
    return pl.pallas_call(
        _proj_body,
        grid=grid,
        in_specs=[
            pl.BlockSpec((BN, H), lambda i: (i, 0)),
            pl.BlockSpec((H, H), lambda i: (0, 0)),
            pl.BlockSpec((H, H), lambda i: (0, 0)),
        ],
        out_specs=[
            pl.BlockSpec((BN, H), lambda i: (i, 0)),
            pl.BlockSpec((BN, H), lambda i: (i, 0)),
        ],
        out_shape=[
            jax.ShapeDtypeStruct((n_pad, H), jnp.float32),
            jax.ShapeDtypeStruct((n_pad, H), jnp.float32),
        ],
        compiler_params=pltpu.CompilerParams(
            dimension_semantics=("arbitrary",)),
    )(node_emb, W1a, W1b)


def kernel(x, edge_index, edge_attr, W_node, b_node, W_edge, b_edge,
           W_msg, b_msg, W_upd, b_upd, W_h1, b_h1, W_h2, b_h2, mask_token):
    mkey = jax.random.key(42)
    k1, k2, k3 = jax.random.split(mkey, 3)
    num_mask = max(1, int(E * MASK_RATIO))
    perm = jax.random.permutation(k1, E)
    mask_indices = perm[:num_mask]
    rand = jax.random.uniform(k2, (num_mask,))
    rand_feat = jax.random.normal(k3, (num_mask, EDGE_IN), dtype=jnp.float32)
    orig_masked = edge_attr[mask_indices]
    token_rep = jnp.broadcast_to(mask_token, (num_mask, EDGE_IN))
    repl = jnp.where(rand[:, None] < 0.8, token_rep,
                     jnp.where(rand[:, None] < 0.9, rand_feat, orig_masked))
    masked_edge_attr = edge_attr.at[mask_indices].set(repl)

    src = edge_index[0]
    dst = edge_index[1]

    h = jax.nn.relu(x @ W_node + b_node)

    b_msg2 = b_msg.reshape(L, 1, H)
    b_upd2 = b_upd.reshape(L, 1, H)
    bedge2 = b_edge.reshape(1, H)

    for l in range(L):
        hs = jnp.take(h, src, axis=0)
        hd = jnp.take(h, dst, axis=0)
        Ws = W_msg[l, :H, :]
        Wd = W_msg[l, H:2 * H, :]
        We = W_msg[l, 2 * H:, :]
        m = _msg_matmul(hs, hd, masked_edge_attr, Ws, Wd, We,
                        b_msg2[l], W_edge, bedge2)
        agg = jax.ops.segment_sum(m, dst, num_segments=N)
        h = _update(h, agg, W_upd[l, :H, :], W_upd[l, H:, :], b_upd2[l], N)

    node_emb = h

    A, B = _proj(node_emb, W_h1[:H, :], W_h1[H:, :], N)
    a_s = jnp.take(A, src, axis=0)
    b_d = jnp.take(B, dst, axis=0)
    reconstructed = _head(a_s, b_d, b_h1.reshape(1, H), W_h2,
                          b_h2.reshape(1, EDGE_IN))

    diff = reconstructed[mask_indices] - edge_attr[mask_indices]
    loss = jnp.mean(diff * diff)
    return (loss, reconstructed, node_emb)


# speedup vs baseline: 1.0557x; 1.0557x over previous
import numpy as np
import jax
import jax.numpy as jnp
from jax.experimental import pallas as pl
from jax.experimental.pallas import tpu as pltpu

N = 50000
E = 800000
H = 128
L = 4
EDGE_IN = 2
MASK_RATIO = 0.15
NUM_MASK = max(1, int(E * MASK_RATIO))

BE = 4000
BN = 2000


def _mask_constants():
    cpu = jax.devices("cpu")[0]
    with jax.default_device(cpu):
        mkey = jax.random.key(42)
        k1, k2, k3 = jax.random.split(mkey, 3)
        perm = jax.random.permutation(k1, E)
        mask_indices = np.asarray(perm[:NUM_MASK])
        rand = np.asarray(jax.random.uniform(k2, (NUM_MASK,)))
        rand_feat = np.asarray(
            jax.random.normal(k3, (NUM_MASK, EDGE_IN), dtype=jnp.float32))
    is_tok = np.zeros((E, 1), np.float32)
    rand_const = np.zeros((E, EDGE_IN), np.float32)
    is_tok[mask_indices[rand < 0.8], 0] = 1.0
    sel = (rand >= 0.8) & (rand < 0.9)
    rand_const[mask_indices[sel]] = rand_feat[sel]
    use_rand = np.zeros((E, 1), np.float32)
    use_rand[mask_indices[sel], 0] = 1.0
    w_loss = np.zeros((E, 1), np.float32)
    w_loss[mask_indices, 0] = 1.0 / (NUM_MASK * EDGE_IN)
    return is_tok, use_rand, rand_const, w_loss


_IS_TOK, _USE_RAND, _RAND_CONST, _W_LOSS = _mask_constants()


def _msg_body(hs_ref, hd_ref, attr_ref, tok_ref, urand_ref, rconst_ref,
              mtok_ref, ws_ref, wd_ref, we_ref, bm_ref,
              wedge_ref, bedge_ref, m_ref):
    attr = attr_ref[...]
    tok = tok_ref[...]
    urand = urand_ref[...]
    masked = (1.0 - tok - urand) * attr + tok * mtok_ref[...] \
        + urand * rconst_ref[...]
    e = jnp.maximum(masked @ wedge_ref[...] + bedge_ref[...], 0.0)
    acc = hs_ref[...] @ ws_ref[...]
    acc = acc + hd_ref[...] @ wd_ref[...]
    acc = acc + e @ we_ref[...]
    m_ref[...] = jnp.maximum(acc + bm_ref[...], 0.0)


def _msg_matmul(hs, hd, attr, mask_tok, Ws, Wd, We, bm, Wedge, bedge):
    grid = (E // BE,)
    ein = lambda i: (i, 0)
    full = lambda i: (0, 0)
    return pl.pallas_call(
        _msg_body,
        grid=grid,
        in_specs=[
            pl.BlockSpec((BE, H), ein),
            pl.BlockSpec((BE, H), ein),
            pl.BlockSpec((BE, EDGE_IN), ein),
            pl.BlockSpec((BE, 1), ein),
            pl.BlockSpec((BE, 1), ein),
            pl.BlockSpec((BE, EDGE_IN), ein),
            pl.BlockSpec((1, EDGE_IN), full),
            pl.BlockSpec((H, H), full),
            pl.BlockSpec((H, H), full),
            pl.BlockSpec((H, H), full),
            pl.BlockSpec((1, H), full),
            pl.BlockSpec((EDGE_IN, H), full),
            pl.BlockSpec((1, H), full),
        ],
        out_specs=pl.BlockSpec((BE, H), ein),
        out_shape=jax.ShapeDtypeStruct((E, H), jnp.float32),
        compiler_params=pltpu.CompilerParams(
            dimension_semantics=("arbitrary",)),
    )(hs, hd, attr, jnp.asarray(_IS_TOK), jnp.asarray(_USE_RAND),
      jnp.asarray(_RAND_CONST), mask_tok.reshape(1, EDGE_IN),
      Ws, Wd, We, bm, Wedge, bedge)


def _upd_body(h_ref, agg_ref, wu1_ref, wu2_ref, bu_ref, out_ref):
    acc = h_ref[...] @ wu1_ref[...]
    acc = acc + agg_ref[...] @ wu2_ref[...]
    out_ref[...] = h_ref[...] + jnp.maximum(acc + bu_ref[...], 0.0)


def _update(h, agg, Wu1, Wu2, bu):
    grid = (N // BN,)
    ein = lambda i: (i, 0)
    full = lambda i: (0, 0)
    return pl.pallas_call(
        _upd_body,
        grid=grid,
        in_specs=[
            pl.BlockSpec((BN, H), ein),
            pl.BlockSpec((BN, H), ein),
            pl.BlockSpec((H, H), full),
            pl.BlockSpec((H, H), full),
            pl.BlockSpec((1, H), full),
        ],
        out_specs=pl.BlockSpec((BN, H), ein),
        out_shape=jax.ShapeDtypeStruct((N, H), jnp.float32),
        compiler_params=pltpu.CompilerParams(
            dimension_semantics=("arbitrary",)),
    )(h, agg, Wu1, Wu2, bu)


def _head_body(a_ref, b_ref, attr_ref, w_ref, b1_ref, w2_ref, b2_ref,
               out_ref, loss_ref):
    i = pl.program_id(0)

    @pl.when(i == 0)
    def _init():
        loss_ref[...] = jnp.zeros_like(loss_ref)

    hid = jnp.maximum(a_ref[...] + b_ref[...] + b1_ref[...], 0.0)
    rec = hid @ w2_ref[...] + b2_ref[...]
    out_ref[...] = rec
    d = rec - attr_ref[...]
    part = jnp.sum(d * d * w_ref[...], axis=0, keepdims=True)
    loss_ref[...] += part


def _head(a_s, b_d, attr, b1, W2, b2):
    grid = (E // BE,)
    ein = lambda i: (i, 0)
    full = lambda i: (0, 0)
    return pl.pallas_call(
        _head_body,
        grid=grid,
        in_specs=[
            pl.BlockSpec((BE, H), ein),
            pl.BlockSpec((BE, H), ein),
            pl.BlockSpec((BE, EDGE_IN), ein),
            pl.BlockSpec((BE, 1), ein),
            pl.BlockSpec((1, H), full),
            pl.BlockSpec((H, EDGE_IN), full),
            pl.BlockSpec((1, EDGE_IN), full),
        ],
        out_specs=[
            pl.BlockSpec((BE, EDGE_IN), ein),
            pl.BlockSpec((1, EDGE_IN), full),
        ],
        out_shape=[
            jax.ShapeDtypeStruct((E, EDGE_IN), jnp.float32),
            jax.ShapeDtypeStruct((1, EDGE_IN), jnp.float32),
        ],
        compiler_params=pltpu.CompilerParams(
            dimension_semantics=("arbitrary",)),
    )(a_s, b_d, attr, jnp.asarray(_W_LOSS), b1, W2, b2)


def _proj_body(h_ref, w1a_ref, w1b_ref, outa_ref, outb_ref):
    outa_ref[...] = h_ref[...] @ w1a_ref[...]
    outb_ref[...] = h_ref[...] @ w1b_ref[...]


def _proj(node_emb, W1a, W1b):
    grid = (N // BN,)
    ein = lambda i: (i, 0)
    full = lambda i: (0, 0)
    return pl.pallas_call(
        _proj_body,
        grid=grid,
        in_specs=[
            pl.BlockSpec((BN, H), ein),
            pl.BlockSpec((H, H), full),
            pl.BlockSpec((H, H), full),
        ],
        out_specs=[
            pl.BlockSpec((BN, H), ein),
            pl.BlockSpec((BN, H), ein),
        ],
        out_shape=[
            jax.ShapeDtypeStruct((N, H), jnp.float32),
            jax.ShapeDtypeStruct((N, H), jnp.float32),
        ],
        compiler_params=pltpu.CompilerParams(
            dimension_semantics=("arbitrary",)),
    )(node_emb, W1a, W1b)


def kernel(x, edge_index, edge_attr, W_node, b_node, W_edge, b_edge,
           W_msg, b_msg, W_upd, b_upd, W_h1, b_h1, W_h2, b_h2, mask_token):
    src = edge_index[0]
    dst = edge_index[1]

    h = jax.nn.relu(x @ W_node + b_node)

    b_msg2 = b_msg.reshape(L, 1, H)
    b_upd2 = b_upd.reshape(L, 1, H)
    bedge2 = b_edge.reshape(1, H)

    for l in range(L):
        hs = jnp.take(h, src, axis=0)
        hd = jnp.take(h, dst, axis=0)
        Ws = W_msg[l, :H, :]
        Wd = W_msg[l, H:2 * H, :]
        We = W_msg[l, 2 * H:, :]
        m = _msg_matmul(hs, hd, edge_attr, mask_token, Ws, Wd, We,
                        b_msg2[l], W_edge, bedge2)
        agg = jax.ops.segment_sum(m, dst, num_segments=N)
        h = _update(h, agg, W_upd[l, :H, :], W_upd[l, H:, :], b_upd2[l])

    node_emb = h

    A, B = _proj(node_emb, W_h1[:H, :], W_h1[H:, :])
    a_s = jnp.take(A, src, axis=0)
    b_d = jnp.take(B, dst, axis=0)
    reconstructed, loss_part = _head(a_s, b_d, edge_attr,
                                     b_h1.reshape(1, H), W_h2,
                                     b_h2.reshape(1, EDGE_IN))
    loss = jnp.sum(loss_part)
    return (loss, reconstructed, node_emb)


# speedup vs baseline: 1.1924x; 1.1295x over previous
import numpy as np
import jax
import jax.numpy as jnp
from jax.experimental import pallas as pl
from jax.experimental.pallas import tpu as pltpu

N = 50000
E = 800000
H = 128
L = 4
EDGE_IN = 2
MASK_RATIO = 0.15
NUM_MASK = max(1, int(E * MASK_RATIO))

BE = 6400
BN = 2000


def _mask_constants():
    cpu = jax.devices("cpu")[0]
    with jax.default_device(cpu):
        mkey = jax.random.key(42)
        k1, k2, k3 = jax.random.split(mkey, 3)
        perm = jax.random.permutation(k1, E)
        mask_indices = np.asarray(perm[:NUM_MASK])
        rand = np.asarray(jax.random.uniform(k2, (NUM_MASK,)))
        rand_feat = np.asarray(
            jax.random.normal(k3, (NUM_MASK, EDGE_IN), dtype=jnp.float32))
    is_tok = np.zeros((E, 1), np.float32)
    rand_const = np.zeros((E, EDGE_IN), np.float32)
    is_tok[mask_indices[rand < 0.8], 0] = 1.0
    sel = (rand >= 0.8) & (rand < 0.9)
    rand_const[mask_indices[sel]] = rand_feat[sel]
    use_rand = np.zeros((E, 1), np.float32)
    use_rand[mask_indices[sel], 0] = 1.0
    w_loss = np.zeros((E, 1), np.float32)
    w_loss[mask_indices, 0] = 1.0 / (NUM_MASK * EDGE_IN)
    return is_tok, use_rand, rand_const, w_loss


_IS_TOK, _USE_RAND, _RAND_CONST, _W_LOSS = _mask_constants()


def _msg_body(hs_ref, hd_ref, attr_ref, tok_ref, urand_ref, rconst_ref,
              mtok_ref, ws_ref, wd_ref, we_ref, bm_ref,
              wedge_ref, bedge_ref, m_ref):
    attr = attr_ref[...]
    tok = tok_ref[...]
    urand = urand_ref[...]
    masked = (1.0 - tok - urand) * attr + tok * mtok_ref[...] \
        + urand * rconst_ref[...]
    e = jnp.maximum(masked @ wedge_ref[...] + bedge_ref[...], 0.0)
    acc = hs_ref[...] @ ws_ref[...]
    acc = acc + hd_ref[...] @ wd_ref[...]
    acc = acc + e @ we_ref[...]
    m_ref[...] = jnp.maximum(acc + bm_ref[...], 0.0)


def _msg_matmul(hs, hd, attr, mask_tok, Ws, Wd, We, bm, Wedge, bedge):
    grid = (E // BE,)
    ein = lambda i: (i, 0)
    full = lambda i: (0, 0)
    return pl.pallas_call(
        _msg_body,
        grid=grid,
        in_specs=[
            pl.BlockSpec((BE, H), ein),
            pl.BlockSpec((BE, H), ein),
            pl.BlockSpec((BE, EDGE_IN), ein),
            pl.BlockSpec((BE, 1), ein),
            pl.BlockSpec((BE, 1), ein),
            pl.BlockSpec((BE, EDGE_IN), ein),
            pl.BlockSpec((1, EDGE_IN), full),
            pl.BlockSpec((H, H), full),
            pl.BlockSpec((H, H), full),
            pl.BlockSpec((H, H), full),
            pl.BlockSpec((1, H), full),
            pl.BlockSpec((EDGE_IN, H), full),
            pl.BlockSpec((1, H), full),
        ],
        out_specs=pl.BlockSpec((BE, H), ein),
        out_shape=jax.ShapeDtypeStruct((E, H), jnp.float32),
        compiler_params=pltpu.CompilerParams(
            dimension_semantics=("arbitrary",)),
    )(hs, hd, attr, jnp.asarray(_IS_TOK), jnp.asarray(_USE_RAND),
      jnp.asarray(_RAND_CONST), mask_tok.reshape(1, EDGE_IN),
      Ws, Wd, We, bm, Wedge, bedge)


def _upd_body(h_ref, agg_ref, wu1_ref, wu2_ref, bu_ref, out_ref):
    acc = h_ref[...] @ wu1_ref[...]
    acc = acc + agg_ref[...] @ wu2_ref[...]
    out_ref[...] = h_ref[...] + jnp.maximum(acc + bu_ref[...], 0.0)


def _update(h, agg, Wu1, Wu2, bu):
    grid = (N // BN,)
    ein = lambda i: (i, 0)
    full = lambda i: (0, 0)
    return pl.pallas_call(
        _upd_body,
        grid=grid,
        in_specs=[
            pl.BlockSpec((BN, H), ein),
            pl.BlockSpec((BN, H), ein),
            pl.BlockSpec((H, H), full),
            pl.BlockSpec((H, H), full),
            pl.BlockSpec((1, H), full),
        ],
        out_specs=pl.BlockSpec((BN, H), ein),
        out_shape=jax.ShapeDtypeStruct((N, H), jnp.float32),
        compiler_params=pltpu.CompilerParams(
            dimension_semantics=("arbitrary",)),
    )(h, agg, Wu1, Wu2, bu)


def _head_body(a_ref, b_ref, attr_ref, w_ref, b1_ref, w2_ref, b2_ref,
               out_ref, loss_ref):
    i = pl.program_id(0)

    @pl.when(i == 0)
    def _init():
        loss_ref[...] = jnp.zeros_like(loss_ref)

    hid = jnp.maximum(a_ref[...] + b_ref[...] + b1_ref[...], 0.0)
    rec = hid @ w2_ref[...] + b2_ref[...]
    out_ref[...] = rec
    d = rec - attr_ref[...]
    part = jnp.sum(d * d * w_ref[...], axis=0, keepdims=True)
    loss_ref[...] += part


def _head(a_s, b_d, attr, b1, W2, b2):
    grid = (E // BE,)
    ein = lambda i: (i, 0)
    full = lambda i: (0, 0)
    return pl.pallas_call(
        _head_body,
        grid=grid,
        in_specs=[
            pl.BlockSpec((BE, H), ein),
            pl.BlockSpec((BE, H), ein),
            pl.BlockSpec((BE, EDGE_IN), ein),
            pl.BlockSpec((BE, 1), ein),
            pl.BlockSpec((1, H), full),
            pl.BlockSpec((H, EDGE_IN), full),
            pl.BlockSpec((1, EDGE_IN), full),
        ],
        out_specs=[
            pl.BlockSpec((BE, EDGE_IN), ein),
            pl.BlockSpec((1, EDGE_IN), full),
        ],
        out_shape=[
            jax.ShapeDtypeStruct((E, EDGE_IN), jnp.float32),
            jax.ShapeDtypeStruct((1, EDGE_IN), jnp.float32),
        ],
        compiler_params=pltpu.CompilerParams(
            dimension_semantics=("arbitrary",)),
    )(a_s, b_d, attr, jnp.asarray(_W_LOSS), b1, W2, b2)


def _proj_body(h_ref, w1a_ref, w1b_ref, outa_ref, outb_ref):
    outa_ref[...] = h_ref[...] @ w1a_ref[...]
    outb_ref[...] = h_ref[...] @ w1b_ref[...]


def _proj(node_emb, W1a, W1b):
    grid = (N // BN,)
    ein = lambda i: (i, 0)
    full = lambda i: (0, 0)
    return pl.pallas_call(
        _proj_body,
        grid=grid,
        in_specs=[
            pl.BlockSpec((BN, H), ein),
            pl.BlockSpec((H, H), full),
            pl.BlockSpec((H, H), full),
        ],
        out_specs=[
            pl.BlockSpec((BN, H), ein),
            pl.BlockSpec((BN, H), ein),
        ],
        out_shape=[
            jax.ShapeDtypeStruct((N, H), jnp.float32),
            jax.ShapeDtypeStruct((N, H), jnp.float32),
        ],
        compiler_params=pltpu.CompilerParams(
            dimension_semantics=("arbitrary",)),
    )(node_emb, W1a, W1b)


def kernel(x, edge_index, edge_attr, W_node, b_node, W_edge, b_edge,
           W_msg, b_msg, W_upd, b_upd, W_h1, b_h1, W_h2, b_h2, mask_token):
    src = edge_index[0]
    dst = edge_index[1]

    h = jax.nn.relu(x @ W_node + b_node)

    b_msg2 = b_msg.reshape(L, 1, H)
    b_upd2 = b_upd.reshape(L, 1, H)
    bedge2 = b_edge.reshape(1, H)

    for l in range(L):
        hs = jnp.take(h, src, axis=0, mode='clip')
        hd = jnp.take(h, dst, axis=0, mode='clip')
        Ws = W_msg[l, :H, :]
        Wd = W_msg[l, H:2 * H, :]
        We = W_msg[l, 2 * H:, :]
        m = _msg_matmul(hs, hd, edge_attr, mask_token, Ws, Wd, We,
                        b_msg2[l], W_edge, bedge2)
        agg = jax.ops.segment_sum(m, dst, num_segments=N)
        h = _update(h, agg, W_upd[l, :H, :], W_upd[l, H:, :], b_upd2[l])

    node_emb = h

    A, B = _proj(node_emb, W_h1[:H, :], W_h1[H:, :])
    a_s = jnp.take(A, src, axis=0, mode='clip')
    b_d = jnp.take(B, dst, axis=0, mode='clip')
    reconstructed, loss_part = _head(a_s, b_d, edge_attr,
                                     b_h1.reshape(1, H), W_h2,
                                     b_h2.reshape(1, EDGE_IN))
    loss = jnp.sum(loss_part)
    return (loss, reconstructed, node_emb)


# speedup vs baseline: 2.2307x; 1.8708x over previous
import numpy as np
from functools import partial
import jax
import jax.numpy as jnp
from jax import lax
from jax.experimental import pallas as pl
from jax.experimental.pallas import tpu as pltpu
from jax.experimental.pallas import tpu_sc as plsc

N = 50000
E = 800000
H = 128
L = 4
EDGE_IN = 2
MASK_RATIO = 0.15
NUM_MASK = max(1, int(E * MASK_RATIO))

BE = 6400
BN = 2000


def _mask_constants():
    cpu = jax.devices("cpu")[0]
    with jax.default_device(cpu):
        mkey = jax.random.key(42)
        k1, k2, k3 = jax.random.split(mkey, 3)
        perm = jax.random.permutation(k1, E)
        mask_indices = np.asarray(perm[:NUM_MASK])
        rand = np.asarray(jax.random.uniform(k2, (NUM_MASK,)))
        rand_feat = np.asarray(
            jax.random.normal(k3, (NUM_MASK, EDGE_IN), dtype=jnp.float32))
    is_tok = np.zeros((E, 1), np.float32)
    rand_const = np.zeros((E, EDGE_IN), np.float32)
    is_tok[mask_indices[rand < 0.8], 0] = 1.0
    sel = (rand >= 0.8) & (rand < 0.9)
    rand_const[mask_indices[sel]] = rand_feat[sel]
    use_rand = np.zeros((E, 1), np.float32)
    use_rand[mask_indices[sel], 0] = 1.0
    w_loss = np.zeros((E, 1), np.float32)
    w_loss[mask_indices, 0] = 1.0 / (NUM_MASK * EDGE_IN)
    return is_tok, use_rand, rand_const, w_loss


_IS_TOK, _USE_RAND, _RAND_CONST, _W_LOSS = _mask_constants()

_NC = 2
_NS = 16
_NW = _NC * _NS
_GC = 400


def _sc_gather_body(nrows, table_hbm, idx_hbm, out_hbm, idx_a, idx_b,
                    rows_v, sem):
    wid = lax.axis_index("s") * _NC + lax.axis_index("c")
    per_w = nrows // _NW
    nchunk = per_w // _GC
    base = wid * per_w

    def _desc(buf):
        idx = idx_a if buf == 0 else idx_b
        return pltpu.make_async_copy(table_hbm.at[idx], rows_v.at[buf],
                                     sem.at[buf])

    def _start(j):
        @pl.when(j % 2 == 0)
        def _():
            pltpu.sync_copy(idx_hbm.at[pl.ds(base + j * _GC, _GC)], idx_a)
            _desc(0).start()

        @pl.when(j % 2 == 1)
        def _():
            pltpu.sync_copy(idx_hbm.at[pl.ds(base + j * _GC, _GC)], idx_b)
            _desc(1).start()

    def _wait(j):
        @pl.when(j % 2 == 0)
        def _():
            _desc(0).wait()

        @pl.when(j % 2 == 1)
        def _():
            _desc(1).wait()

    _start(0)

    def body(j, carry):
        nxt = j + 1

        @pl.when(nxt < nchunk)
        def _():
            _start(nxt)

        _wait(j)
        pltpu.sync_copy(rows_v.at[j % 2],
                        out_hbm.at[pl.ds(base + j * _GC, _GC)])
        return carry

    lax.fori_loop(0, nchunk, body, 0)


def _sc_gather(table, idx, nrows):
    mesh = plsc.VectorSubcoreMesh(core_axis_name="c", subcore_axis_name="s")
    f = partial(
        pl.kernel,
        mesh=mesh,
        out_type=jax.ShapeDtypeStruct((nrows, H), jnp.float32),
        scratch_types=[
            pltpu.VMEM((_GC,), jnp.int32),
            pltpu.VMEM((_GC,), jnp.int32),
            pltpu.VMEM((2, _GC, H), jnp.float32),
            pltpu.SemaphoreType.DMA((2,)),
        ],
    )(partial(_sc_gather_body, nrows))
    return f(table, idx)


def _msg_body(hs_ref, hd_ref, attr_ref, tok_ref, urand_ref, rconst_ref,
              mtok_ref, ws_ref, wd_ref, we_ref, bm_ref,
              wedge_ref, bedge_ref, m_ref):
    attr = attr_ref[...]
    tok = tok_ref[...]
    urand = urand_ref[...]
    masked = (1.0 - tok - urand) * attr + tok * mtok_ref[...] \
        + urand * rconst_ref[...]
    e = jnp.maximum(masked @ wedge_ref[...] + bedge_ref[...], 0.0)
    acc = hs_ref[...] @ ws_ref[...]
    acc = acc + hd_ref[...] @ wd_ref[...]
    acc = acc + e @ we_ref[...]
    m_ref[...] = jnp.maximum(acc + bm_ref[...], 0.0)


def _msg_matmul(gath, attr, mask_tok, Ws, Wd, We, bm, Wedge, bedge):
    grid = (E // BE,)
    nblk = E // BE
    ein = lambda i: (i, 0)
    hd_map = lambda i: (i + nblk, 0)
    full = lambda i: (0, 0)
    return pl.pallas_call(
        _msg_body,
        grid=grid,
        in_specs=[
            pl.BlockSpec((BE, H), ein),
            pl.BlockSpec((BE, H), hd_map),
            pl.BlockSpec((BE, EDGE_IN), ein),
            pl.BlockSpec((BE, 1), ein),
            pl.BlockSpec((BE, 1), ein),
            pl.BlockSpec((BE, EDGE_IN), ein),
            pl.BlockSpec((1, EDGE_IN), full),
            pl.BlockSpec((H, H), full),
            pl.BlockSpec((H, H), full),
            pl.BlockSpec((H, H), full),
            pl.BlockSpec((1, H), full),
            pl.BlockSpec((EDGE_IN, H), full),
            pl.BlockSpec((1, H), full),
        ],
        out_specs=pl.BlockSpec((BE, H), ein),
        out_shape=jax.ShapeDtypeStruct((E, H), jnp.float32),
        compiler_params=pltpu.CompilerParams(
            dimension_semantics=("arbitrary",)),
    )(gath, gath, attr, jnp.asarray(_IS_TOK), jnp.asarray(_USE_RAND),
      jnp.asarray(_RAND_CONST), mask_tok.reshape(1, EDGE_IN),
      Ws, Wd, We, bm, Wedge, bedge)


def _upd_body(h_ref, agg_ref, wu1_ref, wu2_ref, bu_ref, out_ref):
    acc = h_ref[...] @ wu1_ref[...]
    acc = acc + agg_ref[...] @ wu2_ref[...]
    out_ref[...] = h_ref[...] + jnp.maximum(acc + bu_ref[...], 0.0)


def _update(h, agg, Wu1, Wu2, bu):
    grid = (N // BN,)
    ein = lambda i: (i, 0)
    full = lambda i: (0, 0)
    return pl.pallas_call(
        _upd_body,
        grid=grid,
        in_specs=[
            pl.BlockSpec((BN, H), ein),
            pl.BlockSpec((BN, H), ein),
            pl.BlockSpec((H, H), full),
            pl.BlockSpec((H, H), full),
            pl.BlockSpec((1, H), full),
        ],
        out_specs=pl.BlockSpec((BN, H), ein),
        out_shape=jax.ShapeDtypeStruct((N, H), jnp.float32),
        compiler_params=pltpu.CompilerParams(
            dimension_semantics=("arbitrary",)),
    )(h, agg, Wu1, Wu2, bu)


def _head_body(a_ref, b_ref, attr_ref, w_ref, b1_ref, w2_ref, b2_ref,
               out_ref, loss_ref):
    i = pl.program_id(0)

    @pl.when(i == 0)
    def _init():
        loss_ref[...] = jnp.zeros_like(loss_ref)

    hid = jnp.maximum(a_ref[...] + b_ref[...] + b1_ref[...], 0.0)
    rec = hid @ w2_ref[...] + b2_ref[...]
    out_ref[...] = rec
    d = rec - attr_ref[...]
    part = jnp.sum(d * d * w_ref[...], axis=0, keepdims=True)
    loss_ref[...] += part


def _head(gath, attr, b1, W2, b2):
    grid = (E // BE,)
    nblk = E // BE
    ein = lambda i: (i, 0)
    hd_map = lambda i: (i + nblk, 0)
    full = lambda i: (0, 0)
    return pl.pallas_call(
        _head_body,
        grid=grid,
        in_specs=[
            pl.BlockSpec((BE, H), ein),
            pl.BlockSpec((BE, H), hd_map),
            pl.BlockSpec((BE, EDGE_IN), ein),
            pl.BlockSpec((BE, 1), ein),
            pl.BlockSpec((1, H), full),
            pl.BlockSpec((H, EDGE_IN), full),
            pl.BlockSpec((1, EDGE_IN), full),
        ],
        out_specs=[
            pl.BlockSpec((BE, EDGE_IN), ein),
            pl.BlockSpec((1, EDGE_IN), full),
        ],
        out_shape=[
            jax.ShapeDtypeStruct((E, EDGE_IN), jnp.float32),
            jax.ShapeDtypeStruct((1, EDGE_IN), jnp.float32),
        ],
        compiler_params=pltpu.CompilerParams(
            dimension_semantics=("arbitrary",)),
    )(gath, gath, attr, jnp.asarray(_W_LOSS), b1, W2, b2)


def _proj_body(h_ref, w1_ref, out_ref):
    out_ref[...] = h_ref[...] @ w1_ref[0]


def _proj(node_emb, W1ab):
    nblk = N // BN
    grid = (2, nblk)
    return pl.pallas_call(
        _proj_body,
        grid=grid,
        in_specs=[
            pl.BlockSpec((BN, H), lambda g, i: (i, 0)),
            pl.BlockSpec((1, H, H), lambda g, i: (g, 0, 0)),
        ],
        out_specs=pl.BlockSpec((BN, H), lambda g, i: (g * nblk + i, 0)),
        out_shape=jax.ShapeDtypeStruct((2 * N, H), jnp.float32),
        compiler_params=pltpu.CompilerParams(
            dimension_semantics=("arbitrary", "arbitrary")),
    )(node_emb, W1ab)


def kernel(x, edge_index, edge_attr, W_node, b_node, W_edge, b_edge,
           W_msg, b_msg, W_upd, b_upd, W_h1, b_h1, W_h2, b_h2, mask_token):
    src = edge_index[0]
    dst = edge_index[1]
    idx_cat = jnp.concatenate([src, dst])
    idx_head = jnp.concatenate([src, dst + N])

    h = jax.nn.relu(x @ W_node + b_node)

    b_msg2 = b_msg.reshape(L, 1, H)
    b_upd2 = b_upd.reshape(L, 1, H)
    bedge2 = b_edge.reshape(1, H)

    for l in range(L):
        gath = _sc_gather(h, idx_cat, 2 * E)
        Ws = W_msg[l, :H, :]
        Wd = W_msg[l, H:2 * H, :]
        We = W_msg[l, 2 * H:, :]
        m = _msg_matmul(gath, edge_attr, mask_token, Ws, Wd, We,
                        b_msg2[l], W_edge, bedge2)
        agg = jax.ops.segment_sum(m, dst, num_segments=N)
        h = _update(h, agg, W_upd[l, :H, :], W_upd[l, H:, :], b_upd2[l])

    node_emb = h

    projcat = _proj(node_emb, jnp.stack([W_h1[:H, :], W_h1[H:, :]]))
    gath_head = _sc_gather(projcat, idx_head, 2 * E)
    reconstructed, loss_part = _head(gath_head, edge_attr,
                                     b_h1.reshape(1, H), W_h2,
                                     b_h2.reshape(1, EDGE_IN))
    loss = jnp.sum(loss_part)
    return (loss, reconstructed, node_emb)
